# Initial kernel scaffold; baseline (speedup 1.0000x reference)
import jax
import jax.numpy as jnp
from jax.experimental import pallas as pl


def kernel(x, edge_index, W1, b1, g1, be1, W2, b2, g2, be2, W3, b3, g3, be3, Wc, bc):
    raise NotImplementedError("write your pallas kernel here")


# speedup vs baseline: 2.5092x; 2.5092x over previous
import functools

import jax
import jax.numpy as jnp
from jax import lax
from jax.experimental import pallas as pl
from jax.experimental.pallas import tpu as pltpu
from jax.experimental.pallas import tpu_sc as plsc

N_NODES = 10000
NCORES = 2
NSUB = 16
NW = NCORES * NSUB
CHUNK = 128
N_PAD = 10240
HALF = 5120
ACC_ROWS = 8192
TRASH = ACC_ROWS - HALF


def _mesh():
    return plsc.VectorSubcoreMesh(core_axis_name="c", subcore_axis_name="s")


def _prep_edges(edge_index):
    e = edge_index.shape[1]
    c = -(-e // (NW * CHUNK))
    e_pad = NW * c * CHUNK
    pad = e_pad - e
    pad_iota = jnp.arange(pad, dtype=jnp.int32)
    src = jnp.concatenate([edge_index[0], N_NODES + (pad_iota % (N_PAD - N_NODES))])
    dst = jnp.concatenate([edge_index[1], N_NODES + (pad_iota % (N_PAD - N_NODES))])
    return src.reshape(NW, c, CHUNK), dst.reshape(NW, c, CHUNK)


def _remap_half(dst_v, dstl_v, j, half):
    base = half * HALF
    toff = lax.rem(j * CHUNK, TRASH)
    for k in range(CHUNK // 16):
        dv = dst_v[pl.ds(k * 16, 16)]
        local = dv - base
        ok = (local >= 0) & (local < HALF)
        trash = HALF + toff + k * 16 + lax.iota(jnp.int32, 16)
        dstl_v[pl.ds(k * 16, 16)] = jnp.where(ok, local, trash)


def _histogram(idx3):
    c_chunks = idx3.shape[1]
    zeros = jnp.zeros((ACC_ROWS // NSUB, 128), jnp.float32)
    ones = jnp.ones((CHUNK, 128), jnp.float32)

    @functools.partial(
        pl.kernel,
        out_type=jax.ShapeDtypeStruct((NCORES, 2, HALF, 128), jnp.float32),
        mesh=_mesh(),
        scratch_types=[
            pltpu.VMEM((CHUNK,), jnp.int32),
            pltpu.VMEM((CHUNK,), jnp.int32),
            pltpu.VMEM((CHUNK, 128), jnp.float32),
            pltpu.VMEM_SHARED((ACC_ROWS, 128), jnp.float32),
        ],
    )
    def hist_kernel(idx_hbm, zeros_hbm, ones_hbm, out_hbm,
                    idx_v, idxl_v, ones_v, acc):
        c = lax.axis_index("c")
        s = lax.axis_index("s")
        w = c * NSUB + s
        zbase = s * (ACC_ROWS // NSUB)
        obase = s * (HALF // NSUB)
        pltpu.sync_copy(ones_hbm, ones_v)
        for h in range(2):
            pltpu.sync_copy(zeros_hbm,
                            acc.at[pl.ds(zbase, ACC_ROWS // NSUB), :])
            plsc.subcore_barrier()

            def body(j, _):
                pltpu.sync_copy(idx_hbm.at[w, j], idx_v)
                _remap_half(idx_v, idxl_v, j, h)
                pltpu.sync_copy(ones_v, acc.at[idxl_v], add=True)
                return 0
            lax.fori_loop(0, c_chunks, body, 0)
            plsc.subcore_barrier()
            pltpu.sync_copy(
                acc.at[pl.ds(obase, HALF // NSUB), :],
                out_hbm.at[c, h, pl.ds(obase, HALF // NSUB), :])
            plsc.subcore_barrier()

    return hist_kernel(idx3, zeros, ones)


def _propagate(z_wide, src3, dst3):
    c_chunks = src3.shape[1]
    zeros = jnp.zeros((ACC_ROWS // NSUB, 128), jnp.float32)

    @functools.partial(
        pl.kernel,
        out_type=jax.ShapeDtypeStruct((NCORES, 2, HALF, 128), jnp.float32),
        mesh=_mesh(),
        scratch_types=[
            pltpu.VMEM((CHUNK,), jnp.int32),
            pltpu.VMEM((CHUNK,), jnp.int32),
            pltpu.VMEM((CHUNK,), jnp.int32),
            pltpu.VMEM((CHUNK, 128), jnp.float32),
            pltpu.VMEM_SHARED((ACC_ROWS, 128), jnp.float32),
            pltpu.SemaphoreType.DMA,
        ],
    )
    def prop_kernel(z_hbm, src_hbm, dst_hbm, zeros_hbm, out_hbm,
                    src_v, dst_v, dstl_v, rows_v, acc, sem):
        c = lax.axis_index("c")
        s = lax.axis_index("s")
        w = c * NSUB + s
        zbase = s * (ACC_ROWS // NSUB)
        obase = s * (HALF // NSUB)
        for h in range(2):
            pltpu.sync_copy(zeros_hbm,
                            acc.at[pl.ds(zbase, ACC_ROWS // NSUB), :])
            plsc.subcore_barrier()

            def body(j, _):
                pltpu.sync_copy(src_hbm.at[w, j], src_v)
                pltpu.sync_copy(dst_hbm.at[w, j], dst_v)
                _remap_half(dst_v, dstl_v, j, h)
                pltpu.async_copy(z_hbm.at[src_v], rows_v, sem).wait()
                pltpu.sync_copy(rows_v, acc.at[dstl_v], add=True)
                return 0
            lax.fori_loop(0, c_chunks, body, 0)
            plsc.subcore_barrier()
            pltpu.sync_copy(
                acc.at[pl.ds(obase, HALF // NSUB), :],
                out_hbm.at[c, h, pl.ds(obase, HALF // NSUB), :])
            plsc.subcore_barrier()

    return prop_kernel(z_wide, src3, dst3, zeros)


def _pad_rows(z):
    f = z.shape[1]
    return jnp.concatenate(
        [z, jnp.zeros((N_PAD - N_NODES, f), jnp.float32)], axis=0)


def _pad_cols(z):
    n, f = z.shape
    if f == 128:
        return z
    return jnp.concatenate([z, jnp.zeros((n, 128 - f), jnp.float32)], axis=1)


def _stitch(p, f):
    top = p[0, 0, :, :f] + p[1, 0, :, :f]
    bot = p[0, 1, : N_NODES - HALF, :f] + p[1, 1, : N_NODES - HALF, :f]
    return jnp.concatenate([top, bot], axis=0)


def _norms_tc(hs, hd):
    def body(hs_ref, hd_ref, ns_ref, nd_ref):
        od = _stitch(hs_ref[...], 1)
        idg = _stitch(hd_ref[...], 1)
        ns_ref[...] = jnp.where(od > 0, lax.rsqrt(jnp.maximum(od, 1.0)), 0.0)
        nd_ref[...] = jnp.where(idg > 0, lax.rsqrt(jnp.maximum(idg, 1.0)), 0.0)

    return pl.pallas_call(
        body,
        out_shape=(jax.ShapeDtypeStruct((N_NODES, 1), jnp.float32),
                   jax.ShapeDtypeStruct((N_NODES, 1), jnp.float32)),
    )(hs, hd)


def _first_tc(x, w1, ns):
    def body(x_ref, w1_ref, ns_ref, z_ref):
        z = jnp.dot(x_ref[...], w1_ref[...], preferred_element_type=jnp.float32)
        z_ref[...] = _pad_rows(_pad_cols(z * ns_ref[...]))

    return pl.pallas_call(
        body,
        out_shape=jax.ShapeDtypeStruct((N_PAD, 128), jnp.float32),
    )(x, w1, ns)


def _comb_tc(p, nd, b):
    f = b.shape[0]

    def body(p_ref, nd_ref, b_ref, y_ref):
        y_ref[...] = _stitch(p_ref[...], f) * nd_ref[...] + b_ref[...]

    return pl.pallas_call(
        body,
        out_shape=jax.ShapeDtypeStruct((N_NODES, f), jnp.float32),
    )(p, nd, b)


def _bn_lrelu(y, g_ref, be_ref):
    mu = jnp.mean(y, axis=0)
    var = jnp.mean((y - mu) ** 2, axis=0)
    h = g_ref[...] * (y - mu) * lax.rsqrt(var + 1e-5) + be_ref[...]
    return jnp.where(h >= 0, h, 0.2 * h)


def _bnmm_tc(y, g, be, w_next, ns):
    def body(y_ref, g_ref, be_ref, wn_ref, ns_ref, z_ref):
        h = _bn_lrelu(y_ref[...], g_ref, be_ref)
        z = jnp.dot(h, wn_ref[...], preferred_element_type=jnp.float32)
        z_ref[...] = _pad_rows(_pad_cols(z * ns_ref[...]))

    return pl.pallas_call(
        body,
        out_shape=jax.ShapeDtypeStruct((N_PAD, 128), jnp.float32),
    )(y, g, be, w_next, ns)


def _final_tc(y, g, be, wc, bc):
    def body(y_ref, g_ref, be_ref, wc_ref, bc_ref, out_ref):
        h = _bn_lrelu(y_ref[...], g_ref, be_ref)
        hg = jnp.mean(h, axis=0, keepdims=True)
        out_ref[...] = (
            jnp.dot(hg, wc_ref[...], preferred_element_type=jnp.float32)
            + bc_ref[...]
        )

    return pl.pallas_call(
        body,
        out_shape=jax.ShapeDtypeStruct((1, wc.shape[1]), jnp.float32),
    )(y, g, be, wc, bc)


def kernel(x, edge_index, W1, b1, g1, be1, W2, b2, g2, be2, W3, b3, g3, be3,
           Wc, bc):
    src3, dst3 = _prep_edges(edge_index)
    hs = _histogram(src3)
    hd = _histogram(dst3)
    ns, nd = _norms_tc(hs, hd)
    z1 = _first_tc(x, W1, ns)
    p1 = _propagate(z1, src3, dst3)
    y1 = _comb_tc(p1, nd, b1)
    z2 = _bnmm_tc(y1, g1, be1, W2, ns)
    p2 = _propagate(z2, src3, dst3)
    y2 = _comb_tc(p2, nd, b2)
    z3 = _bnmm_tc(y2, g2, be2, W3, ns)
    p3 = _propagate(z3, src3, dst3)
    y3 = _comb_tc(p3, nd, b3)
    return _final_tc(y3, g3, be3, Wc, bc)


# speedup vs baseline: 4.1250x; 1.6440x over previous
import functools

import jax
import jax.numpy as jnp
from jax import lax
from jax.experimental import pallas as pl
from jax.experimental.pallas import tpu as pltpu
from jax.experimental.pallas import tpu_sc as plsc

N_NODES = 10000
NCORES = 2
NSUB = 16
NW = NCORES * NSUB
CHUNK = 128
N_PAD = 10240
HALF = 5120
ACC_ROWS = 8192
TRASH = ACC_ROWS - HALF


def _mesh():
    return plsc.VectorSubcoreMesh(core_axis_name="c", subcore_axis_name="s")


def _prep_edges(edge_index):
    e = edge_index.shape[1]
    c = -(-e // (NW * CHUNK))
    e_pad = NW * c * CHUNK
    pad = e_pad - e
    pad_iota = jnp.arange(pad, dtype=jnp.int32)
    src = jnp.concatenate([edge_index[0], N_NODES + (pad_iota % (N_PAD - N_NODES))])
    dst = jnp.concatenate([edge_index[1], N_NODES + (pad_iota % (N_PAD - N_NODES))])
    return src.reshape(NW, c, CHUNK), dst.reshape(NW, c, CHUNK)


def _remap_half(dst_v, dstl_v, j, half):
    base = half * HALF
    toff = lax.rem(j * CHUNK, TRASH)
    for k in range(CHUNK // 16):
        dv = dst_v[pl.ds(k * 16, 16)]
        local = dv - base
        ok = (local >= 0) & (local < HALF)
        trash = HALF + toff + k * 16 + lax.iota(jnp.int32, 16)
        dstl_v[pl.ds(k * 16, 16)] = jnp.where(ok, local, trash)


def _histogram(idx3):
    c_chunks = idx3.shape[1]
    zeros = jnp.zeros((ACC_ROWS // NSUB, 128), jnp.float32)
    ones = jnp.ones((CHUNK, 128), jnp.float32)

    @functools.partial(
        pl.kernel,
        out_type=jax.ShapeDtypeStruct((NCORES, 2, HALF, 128), jnp.float32),
        mesh=_mesh(),
        scratch_types=[
            pltpu.VMEM((CHUNK,), jnp.int32),
            pltpu.VMEM((CHUNK,), jnp.int32),
            pltpu.VMEM((CHUNK, 128), jnp.float32),
            pltpu.VMEM_SHARED((ACC_ROWS, 128), jnp.float32),
        ],
    )
    def hist_kernel(idx_hbm, zeros_hbm, ones_hbm, out_hbm,
                    idx_v, idxl_v, ones_v, acc):
        c = lax.axis_index("c")
        s = lax.axis_index("s")
        w = c * NSUB + s
        zbase = s * (ACC_ROWS // NSUB)
        obase = s * (HALF // NSUB)
        pltpu.sync_copy(ones_hbm, ones_v)
        for h in range(2):
            pltpu.sync_copy(zeros_hbm,
                            acc.at[pl.ds(zbase, ACC_ROWS // NSUB), :])
            plsc.subcore_barrier()

            def body(j, _):
                pltpu.sync_copy(idx_hbm.at[w, j], idx_v)
                _remap_half(idx_v, idxl_v, j, h)
                pltpu.sync_copy(ones_v, acc.at[idxl_v], add=True)
                return 0
            lax.fori_loop(0, c_chunks, body, 0)
            plsc.subcore_barrier()
            pltpu.sync_copy(
                acc.at[pl.ds(obase, HALF // NSUB), :],
                out_hbm.at[c, h, pl.ds(obase, HALF // NSUB), :])
            plsc.subcore_barrier()

    return hist_kernel(idx3, zeros, ones)


def _propagate(z_wide, src3, dst3):
    c_chunks = src3.shape[1]
    zeros = jnp.zeros((ACC_ROWS // NSUB, 128), jnp.float32)

    @functools.partial(
        pl.kernel,
        out_type=jax.ShapeDtypeStruct((NCORES, 2, HALF, 128), jnp.float32),
        mesh=_mesh(),
        scratch_types=[
            [pltpu.VMEM((CHUNK,), jnp.int32)] * 2,
            [pltpu.VMEM((CHUNK,), jnp.int32)] * 2,
            [pltpu.VMEM((CHUNK,), jnp.int32)] * 2,
            [pltpu.VMEM((CHUNK, 128), jnp.float32)] * 2,
            pltpu.VMEM_SHARED((ACC_ROWS, 128), jnp.float32),
            [pltpu.SemaphoreType.DMA] * 2,
            [pltpu.SemaphoreType.DMA] * 2,
        ],
    )
    def prop_kernel(z_hbm, src_hbm, dst_hbm, zeros_hbm, out_hbm,
                    src_v, dst_v, dstl_v, rows_v, acc, sem_g, sem_s):
        c = lax.axis_index("c")
        s = lax.axis_index("s")
        w = c * NSUB + s
        zbase = s * (ACC_ROWS // NSUB)
        obase = s * (HALF // NSUB)
        n_pairs = c_chunks // 2
        for h in range(2):
            pltpu.sync_copy(zeros_hbm,
                            acc.at[pl.ds(zbase, ACC_ROWS // NSUB), :])
            plsc.subcore_barrier()

            for b in range(2):
                pltpu.sync_copy(src_hbm.at[w, b], src_v[b])
                pltpu.sync_copy(dst_hbm.at[w, b], dst_v[b])
                pltpu.async_copy(z_hbm.at[src_v[b]], rows_v[b], sem_g[b])

            def body(jo, _):
                for b in range(2):
                    j = 2 * jo + b
                    _remap_half(dst_v[b], dstl_v[b], j, h)
                    pltpu.make_async_copy(
                        z_hbm.at[src_v[b]], rows_v[b], sem_g[b]).wait()
                    pltpu.async_copy(rows_v[b], acc.at[dstl_v[b]], sem_s[b],
                                     add=True)

                    @pl.when(jo < n_pairs - 1)
                    def _():
                        pltpu.sync_copy(src_hbm.at[w, j + 2], src_v[b])
                        pltpu.sync_copy(dst_hbm.at[w, j + 2], dst_v[b])
                        pltpu.make_async_copy(
                            rows_v[b], acc.at[dstl_v[b]], sem_s[b]).wait()
                        pltpu.async_copy(z_hbm.at[src_v[b]], rows_v[b],
                                         sem_g[b])
                return 0
            lax.fori_loop(0, n_pairs, body, 0)
            for b in range(2):
                pltpu.make_async_copy(
                    rows_v[b], acc.at[dstl_v[b]], sem_s[b]).wait()
            plsc.subcore_barrier()
            pltpu.sync_copy(
                acc.at[pl.ds(obase, HALF // NSUB), :],
                out_hbm.at[c, h, pl.ds(obase, HALF // NSUB), :])
            plsc.subcore_barrier()

    return prop_kernel(z_wide, src3, dst3, zeros)


def _pad_rows(z):
    f = z.shape[1]
    return jnp.concatenate(
        [z, jnp.zeros((N_PAD - N_NODES, f), jnp.float32)], axis=0)


def _pad_cols(z):
    n, f = z.shape
    if f == 128:
        return z
    return jnp.concatenate([z, jnp.zeros((n, 128 - f), jnp.float32)], axis=1)


def _stitch(p, f):
    top = p[0, 0, :, :f] + p[1, 0, :, :f]
    bot = p[0, 1, : N_NODES - HALF, :f] + p[1, 1, : N_NODES - HALF, :f]
    return jnp.concatenate([top, bot], axis=0)


def _norms_tc(hs, hd):
    def body(hs_ref, hd_ref, ns_ref, nd_ref):
        od = _stitch(hs_ref[...], 1)
        idg = _stitch(hd_ref[...], 1)
        ns_ref[...] = jnp.where(od > 0, lax.rsqrt(jnp.maximum(od, 1.0)), 0.0)
        nd_ref[...] = jnp.where(idg > 0, lax.rsqrt(jnp.maximum(idg, 1.0)), 0.0)

    return pl.pallas_call(
        body,
        out_shape=(jax.ShapeDtypeStruct((N_NODES, 1), jnp.float32),
                   jax.ShapeDtypeStruct((N_NODES, 1), jnp.float32)),
    )(hs, hd)


def _first_tc(x, w1, ns):
    def body(x_ref, w1_ref, ns_ref, z_ref):
        z = jnp.dot(x_ref[...], w1_ref[...], preferred_element_type=jnp.float32)
        z_ref[...] = _pad_rows(_pad_cols(z * ns_ref[...]))

    return pl.pallas_call(
        body,
        out_shape=jax.ShapeDtypeStruct((N_PAD, 128), jnp.float32),
    )(x, w1, ns)


def _comb_tc(p, nd, b):
    f = b.shape[0]

    def body(p_ref, nd_ref, b_ref, y_ref):
        y_ref[...] = _stitch(p_ref[...], f) * nd_ref[...] + b_ref[...]

    return pl.pallas_call(
        body,
        out_shape=jax.ShapeDtypeStruct((N_NODES, f), jnp.float32),
    )(p, nd, b)


def _bn_lrelu(y, g_ref, be_ref):
    mu = jnp.mean(y, axis=0)
    var = jnp.mean((y - mu) ** 2, axis=0)
    h = g_ref[...] * (y - mu) * lax.rsqrt(var + 1e-5) + be_ref[...]
    return jnp.where(h >= 0, h, 0.2 * h)


def _bnmm_tc(y, g, be, w_next, ns):
    def body(y_ref, g_ref, be_ref, wn_ref, ns_ref, z_ref):
        h = _bn_lrelu(y_ref[...], g_ref, be_ref)
        z = jnp.dot(h, wn_ref[...], preferred_element_type=jnp.float32)
        z_ref[...] = _pad_rows(_pad_cols(z * ns_ref[...]))

    return pl.pallas_call(
        body,
        out_shape=jax.ShapeDtypeStruct((N_PAD, 128), jnp.float32),
    )(y, g, be, w_next, ns)


def _final_tc(y, g, be, wc, bc):
    def body(y_ref, g_ref, be_ref, wc_ref, bc_ref, out_ref):
        h = _bn_lrelu(y_ref[...], g_ref, be_ref)
        hg = jnp.mean(h, axis=0, keepdims=True)
        out_ref[...] = (
            jnp.dot(hg, wc_ref[...], preferred_element_type=jnp.float32)
            + bc_ref[...]
        )

    return pl.pallas_call(
        body,
        out_shape=jax.ShapeDtypeStruct((1, wc.shape[1]), jnp.float32),
    )(y, g, be, wc, bc)


def kernel(x, edge_index, W1, b1, g1, be1, W2, b2, g2, be2, W3, b3, g3, be3,
           Wc, bc):
    src3, dst3 = _prep_edges(edge_index)
    hs = _histogram(src3)
    hd = _histogram(dst3)
    ns, nd = _norms_tc(hs, hd)
    z1 = _first_tc(x, W1, ns)
    p1 = _propagate(z1, src3, dst3)
    y1 = _comb_tc(p1, nd, b1)
    z2 = _bnmm_tc(y1, g1, be1, W2, ns)
    p2 = _propagate(z2, src3, dst3)
    y2 = _comb_tc(p2, nd, b2)
    z3 = _bnmm_tc(y2, g2, be2, W3, ns)
    p3 = _propagate(z3, src3, dst3)
    y3 = _comb_tc(p3, nd, b3)
    return _final_tc(y3, g3, be3, Wc, bc)


# speedup vs baseline: 4.6491x; 1.1271x over previous
import functools

import jax
import jax.numpy as jnp
from jax import lax
from jax.experimental import pallas as pl
from jax.experimental.pallas import tpu as pltpu
from jax.experimental.pallas import tpu_sc as plsc

N_NODES = 10000
NCORES = 2
NSUB = 16
NW = NCORES * NSUB
CHUNK = 128
N_PAD = 10240
HALF = 5120
ACC_ROWS = 8192
TRASH = ACC_ROWS - HALF


def _mesh():
    return plsc.VectorSubcoreMesh(core_axis_name="c", subcore_axis_name="s")


def _prep_edges(edge_index):
    e = edge_index.shape[1]
    c = -(-e // (NW * CHUNK))
    e_pad = NW * c * CHUNK
    pad = e_pad - e
    pad_iota = jnp.arange(pad, dtype=jnp.int32)
    src = jnp.concatenate([edge_index[0], N_NODES + (pad_iota % (N_PAD - N_NODES))])
    dst = jnp.concatenate([edge_index[1], N_NODES + (pad_iota % (N_PAD - N_NODES))])
    return src.reshape(NW, c, CHUNK), dst.reshape(NW, c, CHUNK)


def _remap_half(dst_v, dstl_v, j, half):
    base = half * HALF
    toff = lax.rem(j * CHUNK, TRASH)
    for k in range(CHUNK // 16):
        dv = dst_v[pl.ds(k * 16, 16)]
        local = dv - base
        ok = (local >= 0) & (local < HALF)
        trash = HALF + toff + k * 16 + lax.iota(jnp.int32, 16)
        dstl_v[pl.ds(k * 16, 16)] = jnp.where(ok, local, trash)


def _histogram(idx3):
    c_chunks = idx3.shape[1]
    zeros = jnp.zeros((ACC_ROWS // NSUB, 128), jnp.float32)
    ones = jnp.ones((CHUNK, 128), jnp.float32)

    @functools.partial(
        pl.kernel,
        out_type=jax.ShapeDtypeStruct((NCORES, 2, HALF, 128), jnp.float32),
        mesh=_mesh(),
        scratch_types=[
            [pltpu.VMEM((CHUNK,), jnp.int32)] * 2,
            [pltpu.VMEM((CHUNK,), jnp.int32)] * 2,
            pltpu.VMEM((CHUNK, 128), jnp.float32),
            pltpu.VMEM_SHARED((ACC_ROWS, 128), jnp.float32),
            [pltpu.SemaphoreType.DMA] * 2,
        ],
    )
    def hist_kernel(idx_hbm, zeros_hbm, ones_hbm, out_hbm,
                    idx_v, idxl_v, ones_v, acc, sem_s):
        c = lax.axis_index("c")
        s = lax.axis_index("s")
        w = c * NSUB + s
        zbase = s * (ACC_ROWS // NSUB)
        obase = s * (HALF // NSUB)
        n_pairs = c_chunks // 2
        pltpu.sync_copy(ones_hbm, ones_v)
        for h in range(2):
            pltpu.sync_copy(zeros_hbm,
                            acc.at[pl.ds(zbase, ACC_ROWS // NSUB), :])
            plsc.subcore_barrier()

            for b in range(2):
                pltpu.sync_copy(idx_hbm.at[w, b], idx_v[b])

            def body(jo, _):
                for b in range(2):
                    j = 2 * jo + b
                    _remap_half(idx_v[b], idxl_v[b], j, h)
                    pltpu.async_copy(ones_v, acc.at[idxl_v[b]], sem_s[b],
                                     add=True)

                    @pl.when(jo < n_pairs - 1)
                    def _():
                        pltpu.sync_copy(idx_hbm.at[w, j + 2], idx_v[b])

                    pltpu.make_async_copy(
                        ones_v, acc.at[idxl_v[b]], sem_s[b]).wait()
                return 0
            lax.fori_loop(0, n_pairs, body, 0)
            plsc.subcore_barrier()
            pltpu.sync_copy(
                acc.at[pl.ds(obase, HALF // NSUB), :],
                out_hbm.at[c, h, pl.ds(obase, HALF // NSUB), :])
            plsc.subcore_barrier()

    return hist_kernel(idx3, zeros, ones)


def _propagate(z_wide, src3, dst3):
    c_chunks = src3.shape[1]
    zeros = jnp.zeros((ACC_ROWS // NSUB, 128), jnp.float32)

    @functools.partial(
        pl.kernel,
        out_type=jax.ShapeDtypeStruct((NCORES, 2, HALF, 128), jnp.float32),
        mesh=_mesh(),
        scratch_types=[
            [pltpu.VMEM((CHUNK,), jnp.int32)] * 2,
            [pltpu.VMEM((CHUNK,), jnp.int32)] * 2,
            [pltpu.VMEM((CHUNK,), jnp.int32)] * 2,
            [pltpu.VMEM((CHUNK, 128), jnp.float32)] * 2,
            pltpu.VMEM_SHARED((ACC_ROWS, 128), jnp.float32),
            [pltpu.SemaphoreType.DMA] * 2,
            [pltpu.SemaphoreType.DMA] * 2,
        ],
    )
    def prop_kernel(z_hbm, src_hbm, dst_hbm, zeros_hbm, out_hbm,
                    src_v, dst_v, dstl_v, rows_v, acc, sem_g, sem_s):
        c = lax.axis_index("c")
        s = lax.axis_index("s")
        w = c * NSUB + s
        zbase = s * (ACC_ROWS // NSUB)
        obase = s * (HALF // NSUB)
        n_pairs = c_chunks // 2
        for h in range(2):
            pltpu.sync_copy(zeros_hbm,
                            acc.at[pl.ds(zbase, ACC_ROWS // NSUB), :])
            plsc.subcore_barrier()

            for b in range(2):
                pltpu.sync_copy(src_hbm.at[w, b], src_v[b])
                pltpu.sync_copy(dst_hbm.at[w, b], dst_v[b])
                pltpu.async_copy(z_hbm.at[src_v[b]], rows_v[b], sem_g[b])

            def body(jo, _):
                for b in range(2):
                    j = 2 * jo + b
                    _remap_half(dst_v[b], dstl_v[b], j, h)
                    pltpu.make_async_copy(
                        z_hbm.at[src_v[b]], rows_v[b], sem_g[b]).wait()
                    pltpu.async_copy(rows_v[b], acc.at[dstl_v[b]], sem_s[b],
                                     add=True)

                    @pl.when(jo < n_pairs - 1)
                    def _():
                        pltpu.sync_copy(src_hbm.at[w, j + 2], src_v[b])
                        pltpu.sync_copy(dst_hbm.at[w, j + 2], dst_v[b])
                        pltpu.make_async_copy(
                            rows_v[b], acc.at[dstl_v[b]], sem_s[b]).wait()
                        pltpu.async_copy(z_hbm.at[src_v[b]], rows_v[b],
                                         sem_g[b])
                return 0
            lax.fori_loop(0, n_pairs, body, 0)
            for b in range(2):
                pltpu.make_async_copy(
                    rows_v[b], acc.at[dstl_v[b]], sem_s[b]).wait()
            plsc.subcore_barrier()
            pltpu.sync_copy(
                acc.at[pl.ds(obase, HALF // NSUB), :],
                out_hbm.at[c, h, pl.ds(obase, HALF // NSUB), :])
            plsc.subcore_barrier()

    return prop_kernel(z_wide, src3, dst3, zeros)


def _pad_rows(z):
    f = z.shape[1]
    return jnp.concatenate(
        [z, jnp.zeros((N_PAD - N_NODES, f), jnp.float32)], axis=0)


def _pad_cols(z):
    n, f = z.shape
    if f == 128:
        return z
    return jnp.concatenate([z, jnp.zeros((n, 128 - f), jnp.float32)], axis=1)


def _stitch(p, f):
    top = p[0, 0, :, :f] + p[1, 0, :, :f]
    bot = p[0, 1, : N_NODES - HALF, :f] + p[1, 1, : N_NODES - HALF, :f]
    return jnp.concatenate([top, bot], axis=0)


def _norms_tc(hs, hd):
    def body(hs_ref, hd_ref, ns_ref, nd_ref):
        od = _stitch(hs_ref[...], 1)
        idg = _stitch(hd_ref[...], 1)
        ns_ref[...] = jnp.where(od > 0, lax.rsqrt(jnp.maximum(od, 1.0)), 0.0)
        nd_ref[...] = jnp.where(idg > 0, lax.rsqrt(jnp.maximum(idg, 1.0)), 0.0)

    return pl.pallas_call(
        body,
        out_shape=(jax.ShapeDtypeStruct((N_NODES, 1), jnp.float32),
                   jax.ShapeDtypeStruct((N_NODES, 1), jnp.float32)),
    )(hs, hd)


def _first_tc(x, w1, ns):
    def body(x_ref, w1_ref, ns_ref, z_ref):
        z = jnp.dot(x_ref[...], w1_ref[...], preferred_element_type=jnp.float32)
        z_ref[...] = _pad_rows(_pad_cols(z * ns_ref[...]))

    return pl.pallas_call(
        body,
        out_shape=jax.ShapeDtypeStruct((N_PAD, 128), jnp.float32),
    )(x, w1, ns)


def _comb_tc(p, nd, b):
    f = b.shape[0]

    def body(p_ref, nd_ref, b_ref, y_ref):
        y_ref[...] = _stitch(p_ref[...], f) * nd_ref[...] + b_ref[...]

    return pl.pallas_call(
        body,
        out_shape=jax.ShapeDtypeStruct((N_NODES, f), jnp.float32),
    )(p, nd, b)


def _bn_lrelu(y, g_ref, be_ref):
    mu = jnp.mean(y, axis=0)
    var = jnp.mean((y - mu) ** 2, axis=0)
    h = g_ref[...] * (y - mu) * lax.rsqrt(var + 1e-5) + be_ref[...]
    return jnp.where(h >= 0, h, 0.2 * h)


def _bnmm_tc(y, g, be, w_next, ns):
    def body(y_ref, g_ref, be_ref, wn_ref, ns_ref, z_ref):
        h = _bn_lrelu(y_ref[...], g_ref, be_ref)
        z = jnp.dot(h, wn_ref[...], preferred_element_type=jnp.float32)
        z_ref[...] = _pad_rows(_pad_cols(z * ns_ref[...]))

    return pl.pallas_call(
        body,
        out_shape=jax.ShapeDtypeStruct((N_PAD, 128), jnp.float32),
    )(y, g, be, w_next, ns)


def _final_tc(y, g, be, wc, bc):
    def body(y_ref, g_ref, be_ref, wc_ref, bc_ref, out_ref):
        h = _bn_lrelu(y_ref[...], g_ref, be_ref)
        hg = jnp.mean(h, axis=0, keepdims=True)
        out_ref[...] = (
            jnp.dot(hg, wc_ref[...], preferred_element_type=jnp.float32)
            + bc_ref[...]
        )

    return pl.pallas_call(
        body,
        out_shape=jax.ShapeDtypeStruct((1, wc.shape[1]), jnp.float32),
    )(y, g, be, wc, bc)


def kernel(x, edge_index, W1, b1, g1, be1, W2, b2, g2, be2, W3, b3, g3, be3,
           Wc, bc):
    src3, dst3 = _prep_edges(edge_index)
    hs = _histogram(src3)
    hd = _histogram(dst3)
    ns, nd = _norms_tc(hs, hd)
    z1 = _first_tc(x, W1, ns)
    p1 = _propagate(z1, src3, dst3)
    y1 = _comb_tc(p1, nd, b1)
    z2 = _bnmm_tc(y1, g1, be1, W2, ns)
    p2 = _propagate(z2, src3, dst3)
    y2 = _comb_tc(p2, nd, b2)
    z3 = _bnmm_tc(y2, g2, be2, W3, ns)
    p3 = _propagate(z3, src3, dst3)
    y3 = _comb_tc(p3, nd, b3)
    return _final_tc(y3, g3, be3, Wc, bc)


# speedup vs baseline: 5.0750x; 1.0916x over previous
import functools

import jax
import jax.numpy as jnp
from jax import lax
from jax.experimental import pallas as pl
from jax.experimental.pallas import tpu as pltpu
from jax.experimental.pallas import tpu_sc as plsc

N_NODES = 10000
NCORES = 2
NSUB = 16
NW = NCORES * NSUB
CHUNK = 128
N_PAD = 10240
HALF = 5120
ACC_ROWS = 8192
TRASH = ACC_ROWS - HALF


def _mesh():
    return plsc.VectorSubcoreMesh(core_axis_name="c", subcore_axis_name="s")


def _prep_edges(edge_index):
    e = edge_index.shape[1]
    c = -(-e // (NW * CHUNK))
    e_pad = NW * c * CHUNK
    pad = e_pad - e
    pad_iota = jnp.arange(pad, dtype=jnp.int32)
    src = jnp.concatenate([edge_index[0], N_NODES + (pad_iota % (N_PAD - N_NODES))])
    dst = jnp.concatenate([edge_index[1], N_NODES + (pad_iota % (N_PAD - N_NODES))])
    return src.reshape(NW, c, CHUNK), dst.reshape(NW, c, CHUNK)


def _remap_half(dst_v, dstl_v, j, half, row=None):
    base = half * HALF
    toff = lax.rem(j * CHUNK, TRASH)
    for k in range(CHUNK // 16):
        sl = pl.ds(k * 16, 16)
        dv = dst_v[sl] if row is None else dst_v[row, sl]
        local = dv - base
        ok = (local >= 0) & (local < HALF)
        trash = HALF + toff + k * 16 + lax.iota(jnp.int32, 16)
        out = jnp.where(ok, local, trash)
        dstl_v[sl] = jnp.clip(out, 0, ACC_ROWS - 1)


def _histogram(idx3):
    c_chunks = idx3.shape[1]
    zeros = jnp.zeros((ACC_ROWS // NSUB, 128), jnp.float32)
    ones = jnp.ones((CHUNK, 128), jnp.float32)

    @functools.partial(
        pl.kernel,
        out_type=jax.ShapeDtypeStruct((NCORES, 2, HALF, 128), jnp.float32),
        mesh=_mesh(),
        scratch_types=[
            pltpu.VMEM((c_chunks, CHUNK), jnp.int32),
            [pltpu.VMEM((CHUNK,), jnp.int32)] * 2,
            pltpu.VMEM((CHUNK, 128), jnp.float32),
            pltpu.VMEM_SHARED((ACC_ROWS, 128), jnp.float32),
            [pltpu.SemaphoreType.DMA] * 2,
        ],
    )
    def hist_kernel(idx_hbm, zeros_hbm, ones_hbm, out_hbm,
                    idx_all, idxl_v, ones_v, acc, sem_s):
        c = lax.axis_index("c")
        s = lax.axis_index("s")
        w = c * NSUB + s
        zbase = s * (ACC_ROWS // NSUB)
        obase = s * (HALF // NSUB)
        n_pairs = c_chunks // 2
        pltpu.sync_copy(ones_hbm, ones_v)
        pltpu.sync_copy(idx_hbm.at[w], idx_all)
        for h in range(2):
            pltpu.sync_copy(zeros_hbm,
                            acc.at[pl.ds(zbase, ACC_ROWS // NSUB), :])
            plsc.subcore_barrier()

            def body(jo, _):
                for b in range(2):
                    j = 2 * jo + b

                    @pl.when(jo > 0)
                    def _():
                        pltpu.make_async_copy(
                            ones_v, acc.at[idxl_v[b]], sem_s[b]).wait()

                    _remap_half(idx_all, idxl_v[b], j, h, row=j)
                    pltpu.async_copy(ones_v, acc.at[idxl_v[b]], sem_s[b],
                                     add=True)
                return 0
            lax.fori_loop(0, n_pairs, body, 0)
            for b in range(2):
                pltpu.make_async_copy(
                    ones_v, acc.at[idxl_v[b]], sem_s[b]).wait()
            plsc.subcore_barrier()
            pltpu.sync_copy(
                acc.at[pl.ds(obase, HALF // NSUB), :],
                out_hbm.at[c, h, pl.ds(obase, HALF // NSUB), :])
            plsc.subcore_barrier()

    return hist_kernel(idx3, zeros, ones)


def _propagate(z_wide, src3, dst3):
    c_chunks = src3.shape[1]
    zeros = jnp.zeros((ACC_ROWS // NSUB, 128), jnp.float32)

    @functools.partial(
        pl.kernel,
        out_type=jax.ShapeDtypeStruct((NCORES, 2, HALF, 128), jnp.float32),
        mesh=_mesh(),
        scratch_types=[
            pltpu.VMEM((c_chunks, CHUNK), jnp.int32),
            pltpu.VMEM((c_chunks, CHUNK), jnp.int32),
            [pltpu.VMEM((CHUNK,), jnp.int32)] * 2,
            [pltpu.VMEM((CHUNK, 128), jnp.float32)] * 2,
            pltpu.VMEM_SHARED((ACC_ROWS, 128), jnp.float32),
            [pltpu.SemaphoreType.DMA] * 2,
            [pltpu.SemaphoreType.DMA] * 2,
        ],
    )
    def prop_kernel(z_hbm, src_hbm, dst_hbm, zeros_hbm, out_hbm,
                    src_all, dst_all, dstl_v, rows_v, acc, sem_g, sem_s):
        c = lax.axis_index("c")
        s = lax.axis_index("s")
        w = c * NSUB + s
        zbase = s * (ACC_ROWS // NSUB)
        obase = s * (HALF // NSUB)
        n_pairs = c_chunks // 2
        pltpu.sync_copy(src_hbm.at[w], src_all)
        pltpu.sync_copy(dst_hbm.at[w], dst_all)
        for h in range(2):
            pltpu.sync_copy(zeros_hbm,
                            acc.at[pl.ds(zbase, ACC_ROWS // NSUB), :])
            plsc.subcore_barrier()

            for b in range(2):
                pltpu.async_copy(z_hbm.at[src_all.at[b]], rows_v[b], sem_g[b])

            def body(jo, _):
                for b in range(2):
                    j = 2 * jo + b
                    _remap_half(dst_all, dstl_v[b], j, h, row=j)
                    pltpu.make_async_copy(
                        z_hbm.at[src_all.at[j]], rows_v[b], sem_g[b]).wait()
                    pltpu.async_copy(rows_v[b], acc.at[dstl_v[b]], sem_s[b],
                                     add=True)

                    @pl.when(jo < n_pairs - 1)
                    def _():
                        pltpu.make_async_copy(
                            rows_v[b], acc.at[dstl_v[b]], sem_s[b]).wait()
                        pltpu.async_copy(z_hbm.at[src_all.at[j + 2]],
                                         rows_v[b], sem_g[b])
                return 0
            lax.fori_loop(0, n_pairs, body, 0)
            for b in range(2):
                pltpu.make_async_copy(
                    rows_v[b], acc.at[dstl_v[b]], sem_s[b]).wait()
            plsc.subcore_barrier()
            pltpu.sync_copy(
                acc.at[pl.ds(obase, HALF // NSUB), :],
                out_hbm.at[c, h, pl.ds(obase, HALF // NSUB), :])
            plsc.subcore_barrier()

    return prop_kernel(z_wide, src3, dst3, zeros)


def _pad_rows(z):
    f = z.shape[1]
    return jnp.concatenate(
        [z, jnp.zeros((N_PAD - N_NODES, f), jnp.float32)], axis=0)


def _pad_cols(z):
    n, f = z.shape
    if f == 128:
        return z
    return jnp.concatenate([z, jnp.zeros((n, 128 - f), jnp.float32)], axis=1)


def _stitch(p, f):
    top = p[0, 0, :, :f] + p[1, 0, :, :f]
    bot = p[0, 1, : N_NODES - HALF, :f] + p[1, 1, : N_NODES - HALF, :f]
    return jnp.concatenate([top, bot], axis=0)


def _norms_tc(hs, hd):
    def body(hs_ref, hd_ref, ns_ref, nd_ref):
        od = _stitch(hs_ref[...], 1)
        idg = _stitch(hd_ref[...], 1)
        ns_ref[...] = jnp.where(od > 0, lax.rsqrt(jnp.maximum(od, 1.0)), 0.0)
        nd_ref[...] = jnp.where(idg > 0, lax.rsqrt(jnp.maximum(idg, 1.0)), 0.0)

    return pl.pallas_call(
        body,
        out_shape=(jax.ShapeDtypeStruct((N_NODES, 1), jnp.float32),
                   jax.ShapeDtypeStruct((N_NODES, 1), jnp.float32)),
    )(hs, hd)


def _first_tc(x, w1, ns):
    def body(x_ref, w1_ref, ns_ref, z_ref):
        z = jnp.dot(x_ref[...], w1_ref[...], preferred_element_type=jnp.float32)
        z_ref[...] = _pad_rows(_pad_cols(z * ns_ref[...]))

    return pl.pallas_call(
        body,
        out_shape=jax.ShapeDtypeStruct((N_PAD, 128), jnp.float32),
    )(x, w1, ns)


def _comb_tc(p, nd, b):
    f = b.shape[0]

    def body(p_ref, nd_ref, b_ref, y_ref):
        y_ref[...] = _stitch(p_ref[...], f) * nd_ref[...] + b_ref[...]

    return pl.pallas_call(
        body,
        out_shape=jax.ShapeDtypeStruct((N_NODES, f), jnp.float32),
    )(p, nd, b)


def _bn_lrelu(y, g_ref, be_ref):
    mu = jnp.mean(y, axis=0)
    var = jnp.mean((y - mu) ** 2, axis=0)
    h = g_ref[...] * (y - mu) * lax.rsqrt(var + 1e-5) + be_ref[...]
    return jnp.where(h >= 0, h, 0.2 * h)


def _bnmm_tc(y, g, be, w_next, ns):
    def body(y_ref, g_ref, be_ref, wn_ref, ns_ref, z_ref):
        h = _bn_lrelu(y_ref[...], g_ref, be_ref)
        z = jnp.dot(h, wn_ref[...], preferred_element_type=jnp.float32)
        z_ref[...] = _pad_rows(_pad_cols(z * ns_ref[...]))

    return pl.pallas_call(
        body,
        out_shape=jax.ShapeDtypeStruct((N_PAD, 128), jnp.float32),
    )(y, g, be, w_next, ns)


def _final_tc(y, g, be, wc, bc):
    def body(y_ref, g_ref, be_ref, wc_ref, bc_ref, out_ref):
        h = _bn_lrelu(y_ref[...], g_ref, be_ref)
        hg = jnp.mean(h, axis=0, keepdims=True)
        out_ref[...] = (
            jnp.dot(hg, wc_ref[...], preferred_element_type=jnp.float32)
            + bc_ref[...]
        )

    return pl.pallas_call(
        body,
        out_shape=jax.ShapeDtypeStruct((1, wc.shape[1]), jnp.float32),
    )(y, g, be, wc, bc)


def kernel(x, edge_index, W1, b1, g1, be1, W2, b2, g2, be2, W3, b3, g3, be3,
           Wc, bc):
    src3, dst3 = _prep_edges(edge_index)
    hs = _histogram(src3)
    hd = _histogram(dst3)
    ns, nd = _norms_tc(hs, hd)
    z1 = _first_tc(x, W1, ns)
    p1 = _propagate(z1, src3, dst3)
    y1 = _comb_tc(p1, nd, b1)
    z2 = _bnmm_tc(y1, g1, be1, W2, ns)
    p2 = _propagate(z2, src3, dst3)
    y2 = _comb_tc(p2, nd, b2)
    z3 = _bnmm_tc(y2, g2, be2, W3, ns)
    p3 = _propagate(z3, src3, dst3)
    y3 = _comb_tc(p3, nd, b3)
    return _final_tc(y3, g3, be3, Wc, bc)
